# baseline (speedup 1.0000x reference)
import functools

import jax
import jax.numpy as jnp
from jax import lax
from jax.experimental import pallas as pl
from jax.experimental.pallas import tpu as pltpu
from jax.experimental.pallas import tpu_sc as plsc

B = 4096
T = 64
D = 16
K_RANK = 204
NBUCKET = 256
L = 16
SLAB = B * D
EPS = 1e-8
TOPBIT = -(2 ** 31)


def _sc_var_kernel(xf_hbm, xr_hbm, var_hbm, buf, histf, idxtab, stage):
    c = lax.axis_index("c")
    s = lax.axis_index("s")
    wid = s * 2 + c

    lanes = lax.broadcasted_iota(jnp.int32, (L,), 0)
    ones = jnp.ones((L,), jnp.int32)
    zeros_i = jnp.zeros((L,), jnp.int32)
    lane_base = (
        lax.shift_right_logical(lanes, jnp.full((L,), 3, jnp.int32)) * (SLAB // 2)
        + lax.bitwise_and(lanes, jnp.full((L,), 7, jnp.int32)) * 128)

    def zero_hist(b, carry):
        histf[pl.ds(b * L, L)] = zeros_i
        return carry

    lax.fori_loop(0, NBUCKET, zero_hist, 0)

    def mk_idx(n0, carry):
        cvec = lax.bitwise_and(lanes + n0, jnp.full((L,), 127, jnp.int32))
        idxtab[n0] = lane_base + cvec
        return carry

    lax.fori_loop(0, 128, mk_idx, 0)

    def hist_pass(shift_dig, pfx, convert, n_lo=0, n_hi=B):
        @plsc.parallel_loop(n_lo, n_hi, 1, unroll=16)
        def _(n):
            ivec = idxtab[lax.bitwise_and(n, 127)] + lax.shift_right_logical(n, 7) * 1024
            got = plsc.load_gather(buf, [ivec])
            if convert:
                sbits = plsc.bitcast(got, jnp.int32)
                m = lax.shift_right_arithmetic(sbits, jnp.full((L,), 31, jnp.int32))
                key = lax.bitwise_xor(
                    sbits, lax.bitwise_or(m, jnp.full((L,), TOPBIT, jnp.int32)))
                plsc.store_scatter(buf, [ivec], plsc.bitcast(key, jnp.float32))
            else:
                key = plsc.bitcast(got, jnp.int32)
            if shift_dig >= 4:
                hrow = lax.bitwise_and(
                    lax.shift_right_logical(
                        key, jnp.full((L,), shift_dig - 4, jnp.int32)),
                    jnp.full((L,), 0xFF0, jnp.int32))
            else:
                hrow = lax.bitwise_and(
                    lax.shift_left(key, jnp.full((L,), 4, jnp.int32)),
                    jnp.full((L,), 0xFF0, jnp.int32))
            if pfx is None:
                plsc.addupdate_scatter(histf, [hrow + lanes], ones)
            else:
                hi = lax.shift_right_logical(
                    key, jnp.full((L,), shift_dig + 8, jnp.int32))
                plsc.addupdate_scatter(histf, [hrow + lanes], ones,
                                       mask=(hi == pfx))

    def bucket_scan(kvec):
        UB = 8

        def bs(bb, carry):
            cum, bucket, kbase = carry
            base = bb * UB
            for u in range(UB):
                b = base + u
                h = histf[pl.ds(b * L, L)]
                histf[pl.ds(b * L, L)] = zeros_i
                ncum = cum + h
                newly = jnp.logical_and(cum <= kvec, ncum > kvec)
                bucket = jnp.where(newly, b, bucket)
                kbase = jnp.where(newly, cum, kbase)
                cum = ncum
            return (cum, bucket, kbase)

        cum, bucket, kbase = lax.fori_loop(
            0, NBUCKET // UB, bs, (zeros_i, zeros_i, zeros_i))
        return bucket, kvec - kbase

    def do_task(x_hbm, arr_idx, t):
        pltpu.sync_copy(x_hbm.at[t], buf)
        kvec = jnp.full((L,), K_RANK, jnp.int32)

        hist_pass(24, None, True)
        b0, kvec = bucket_scan(kvec)
        pfx = b0

        for shift_dig in (16, 8, 0):
            hist_pass(shift_dig, pfx, False)
            bnext, kvec = bucket_scan(kvec)
            pfx = lax.bitwise_or(
                lax.shift_left(pfx, jnp.full((L,), 8, jnp.int32)), bnext)

        neg = pfx >= 0
        u = jnp.where(
            neg, ~pfx, lax.bitwise_xor(pfx, jnp.full((L,), TOPBIT, jnp.int32)))
        stage[...] = plsc.bitcast(u, jnp.float32)
        pltpu.sync_copy(stage, var_hbm.at[arr_idx, t])

    for j in range(2):
        t = wid * 2 + j
        do_task(xf_hbm, 0, t)
        do_task(xr_hbm, 1, t)


def _as_slabs(x):
    return (x.transpose(1, 2, 0)
             .reshape(T, 2, 8, B // 128, 128)
             .transpose(0, 1, 3, 2, 4)
             .reshape(T, SLAB))


@jax.jit
def _sc_var(x_fake, x_real):
    zf, zr = _as_slabs(x_fake), _as_slabs(x_real)
    mesh = plsc.VectorSubcoreMesh(core_axis_name="c", subcore_axis_name="s")
    return pl.kernel(
        _sc_var_kernel,
        out_type=jax.ShapeDtypeStruct((2, T, D), jnp.float32),
        mesh=mesh,
        scratch_types=[
            pltpu.VMEM((SLAB,), jnp.float32),
            pltpu.VMEM((NBUCKET * L,), jnp.int32),
            pltpu.VMEM((128, L), jnp.int32),
            pltpu.VMEM((L,), jnp.float32),
        ],
        compiler_params=pltpu.CompilerParams(
            needs_layout_passes=False, use_tc_tiling_on_sc=False),
    )(zf, zr)


def _tc_loss_kernel(v_ref, abs_ref, rel_ref):
    v = v_ref[...]
    d = jnp.abs(v[0:1, :] - v[1:2, :])
    vr = jnp.abs(v[1:2, :])
    inv_n = 1.0 / (T * D)
    abs_ref[0, 0] = jnp.sum(d) * inv_n
    rel_ref[0, 0] = jnp.sum(d / (vr + EPS)) * inv_n


@jax.jit
def _tc_loss(var2):
    return pl.pallas_call(
        _tc_loss_kernel,
        out_shape=(
            jax.ShapeDtypeStruct((1, 1), jnp.float32),
            jax.ShapeDtypeStruct((1, 1), jnp.float32),
        ),
        out_specs=(
            pl.BlockSpec(memory_space=pltpu.SMEM),
            pl.BlockSpec(memory_space=pltpu.SMEM),
        ),
    )(var2)


def kernel(x_fake, x_real):
    var = _sc_var(x_fake, x_real)
    abs_l, rel_l = _tc_loss(var.reshape(2, T * D))
    return (abs_l[0, 0], rel_l[0, 0])


# speedup vs baseline: 1.0393x; 1.0393x over previous
import functools

import jax
import jax.numpy as jnp
from jax import lax
from jax.experimental import pallas as pl
from jax.experimental.pallas import tpu as pltpu
from jax.experimental.pallas import tpu_sc as plsc

B = 4096
T = 64
D = 16
K_RANK = 204
NBUCKET = 256
L = 16
SLAB = B * D
EPS = 1e-8
TOPBIT = -(2 ** 31)


def _sc_var_kernel(xf_hbm, xr_hbm, var_hbm, buf, hist, idxtab, stage):
    c = lax.axis_index("c")
    s = lax.axis_index("s")
    wid = s * 2 + c

    lanes = lax.broadcasted_iota(jnp.int32, (L,), 0)
    ones = jnp.ones((L,), jnp.int32)
    zeros_i = jnp.zeros((L,), jnp.int32)
    lane_base = (
        lax.shift_right_logical(lanes, jnp.full((L,), 3, jnp.int32)) * (SLAB // 2)
        + lax.bitwise_and(lanes, jnp.full((L,), 7, jnp.int32)) * 128)

    def zero_hist(b, carry):
        hist[b] = zeros_i
        return carry

    lax.fori_loop(0, NBUCKET, zero_hist, 0)

    def mk_idx(n0, carry):
        cvec = lax.bitwise_and(lanes + n0, jnp.full((L,), 127, jnp.int32))
        idxtab[n0] = lane_base + cvec
        return carry

    lax.fori_loop(0, 128, mk_idx, 0)

    def hist_pass(shift_dig, pfx, convert, n_lo=0, n_hi=B):
        @plsc.parallel_loop(n_lo, n_hi, 1, unroll=16)
        def _(n):
            ivec = idxtab[lax.bitwise_and(n, 127)] + lax.shift_right_logical(n, 7) * 1024
            got = plsc.load_gather(buf, [ivec])
            if convert:
                sbits = plsc.bitcast(got, jnp.int32)
                m = lax.shift_right_arithmetic(sbits, jnp.full((L,), 31, jnp.int32))
                key = lax.bitwise_xor(
                    sbits, lax.bitwise_or(m, jnp.full((L,), TOPBIT, jnp.int32)))
                plsc.store_scatter(buf, [ivec], plsc.bitcast(key, jnp.float32))
            else:
                key = plsc.bitcast(got, jnp.int32)
            d = lax.bitwise_and(
                lax.shift_right_logical(key, jnp.full((L,), shift_dig, jnp.int32)),
                jnp.full((L,), 255, jnp.int32))
            if pfx is None:
                plsc.addupdate_scatter(hist, [d, lanes], ones)
            else:
                hi = lax.shift_right_logical(
                    key, jnp.full((L,), shift_dig + 8, jnp.int32))
                plsc.addupdate_scatter(hist, [d, lanes], ones, mask=(hi == pfx))

    def bucket_scan(kvec):
        UB = 8

        def bs(bb, carry):
            cum, bucket, kbase = carry
            base = bb * UB
            for u in range(UB):
                b = base + u
                h = hist[b]
                hist[b] = zeros_i
                ncum = cum + h
                newly = jnp.logical_and(cum <= kvec, ncum > kvec)
                bucket = jnp.where(newly, b, bucket)
                kbase = jnp.where(newly, cum, kbase)
                cum = ncum
            return (cum, bucket, kbase)

        cum, bucket, kbase = lax.fori_loop(
            0, NBUCKET // UB, bs, (zeros_i, zeros_i, zeros_i))
        return bucket, kvec - kbase

    def do_task(x_hbm, arr_idx, t):
        pltpu.sync_copy(x_hbm.at[t], buf)
        kvec = jnp.full((L,), K_RANK, jnp.int32)

        hist_pass(24, None, True)
        b0, kvec = bucket_scan(kvec)
        pfx = b0

        for shift_dig in (16, 8, 0):
            hist_pass(shift_dig, pfx, False)
            bnext, kvec = bucket_scan(kvec)
            pfx = lax.bitwise_or(
                lax.shift_left(pfx, jnp.full((L,), 8, jnp.int32)), bnext)

        neg = pfx >= 0
        u = jnp.where(
            neg, ~pfx, lax.bitwise_xor(pfx, jnp.full((L,), TOPBIT, jnp.int32)))
        stage[...] = plsc.bitcast(u, jnp.float32)
        pltpu.sync_copy(stage, var_hbm.at[arr_idx, t])

    for j in range(2):
        t = wid * 2 + j
        do_task(xf_hbm, 0, t)
        do_task(xr_hbm, 1, t)


def _as_slabs(x):
    return (x.transpose(1, 2, 0)
             .reshape(T, 2, 8, B // 128, 128)
             .transpose(0, 1, 3, 2, 4)
             .reshape(T, SLAB))


@jax.jit
def _sc_var(x_fake, x_real):
    zf, zr = _as_slabs(x_fake), _as_slabs(x_real)
    mesh = plsc.VectorSubcoreMesh(core_axis_name="c", subcore_axis_name="s")
    return pl.kernel(
        _sc_var_kernel,
        out_type=jax.ShapeDtypeStruct((2, T, D), jnp.float32),
        mesh=mesh,
        scratch_types=[
            pltpu.VMEM((SLAB,), jnp.float32),
            pltpu.VMEM((NBUCKET, L), jnp.int32),
            pltpu.VMEM((128, L), jnp.int32),
            pltpu.VMEM((L,), jnp.float32),
        ],
        compiler_params=pltpu.CompilerParams(
            needs_layout_passes=False, use_tc_tiling_on_sc=False),
    )(zf, zr)


def _tc_loss_kernel(v_ref, abs_ref, rel_ref):
    v = v_ref[...]
    d = jnp.abs(v[0:1, :] - v[1:2, :])
    vr = jnp.abs(v[1:2, :])
    inv_n = 1.0 / (T * D)
    abs_ref[0, 0] = jnp.sum(d) * inv_n
    rel_ref[0, 0] = jnp.sum(d / (vr + EPS)) * inv_n


@jax.jit
def _tc_loss(var2):
    return pl.pallas_call(
        _tc_loss_kernel,
        out_shape=(
            jax.ShapeDtypeStruct((1, 1), jnp.float32),
            jax.ShapeDtypeStruct((1, 1), jnp.float32),
        ),
        out_specs=(
            pl.BlockSpec(memory_space=pltpu.SMEM),
            pl.BlockSpec(memory_space=pltpu.SMEM),
        ),
    )(var2)


def kernel(x_fake, x_real):
    var = _sc_var(x_fake, x_real)
    abs_l, rel_l = _tc_loss(var.reshape(2, T * D))
    return (abs_l[0, 0], rel_l[0, 0])
